# baseline (speedup 1.0000x reference)
import functools

import jax
import jax.numpy as jnp
from jax import lax
from jax.experimental import pallas as pl
from jax.experimental.pallas import tpu as pltpu
from jax.experimental.pallas import tpu_sc as plsc

NC = 2
NS = 16
L = 16
NW = NC * NS
TW = 128
STG = 256
PAD = 16

_CPARAMS = pltpu.CompilerParams(
    needs_layout_passes=False, use_tc_tiling_on_sc=True
)
_MESH = plsc.VectorSubcoreMesh(core_axis_name="c", subcore_axis_name="s")


def _make_extract(B, N, D):
    n_slab = (N + TW - 1) // TW
    base_per_w = n_slab // NW
    extra = n_slab - base_per_w * NW
    nvec = B // L

    @functools.partial(
        pl.kernel,
        out_type=jax.ShapeDtypeStruct((B + PAD, TW), jnp.float32),
        mesh=_MESH,
        compiler_params=_CPARAMS,
        scratch_types=[
            pltpu.VMEM((B,), jnp.int32),
            pltpu.VMEM((B,), jnp.int32),
            pltpu.VMEM((B,), jnp.int32),
            pltpu.VMEM((2, D, TW), jnp.float32),
            pltpu.VMEM((STG, TW), jnp.float32),
            pltpu.VMEM((STG,), jnp.int32),
            pltpu.SemaphoreType.DMA,
            pltpu.SemaphoreType.DMA,
            pltpu.SemaphoreType.DMA,
        ],
    )
    def kern(idx_h, tabT, rows_h, idx_v, er_v, eb_v, slab, stg, stgb,
             sem0, sem1, sem2):
        wid = lax.axis_index("s") * NC + lax.axis_index("c")
        my_n = base_per_w + jnp.where(wid < extra, 1, 0)
        slab0 = wid * base_per_w + jnp.minimum(wid, extra)
        lo = slab0 * TW
        sems = (sem0, sem1)

        pltpu.sync_copy(idx_h, idx_v)
        lane_iota = lax.iota(jnp.int32, L)

        hi = lo + my_n * TW
        def collect(v, cnt):
            r = idx_v[pl.ds(v * L, L)]
            b = lane_iota + v * L
            m = (r >= lo) & (r < hi)
            plsc.store_compressed(er_v.at[pl.ds(cnt, L)], r, mask=m)
            plsc.store_compressed(eb_v.at[pl.ds(cnt, L)], b, mask=m)
            nh = lax.reduce_max(plsc.all_reduce_population_count(m), (0,))
            return cnt + nh
        cnt = lax.fori_loop(0, nvec, collect, jnp.int32(0))
        nev = (cnt + (L - 1)) // L

        def fire(c, buf):
            start = pl.multiple_of((slab0 + c) * TW, TW)
            pltpu.async_copy(tabT.at[:, pl.ds(start, TW)], slab.at[buf],
                             sems[buf])

        @pl.when(my_n > 0)
        def _():
            fire(0, 0)

        @pl.when(my_n > 1)
        def _():
            fire(1, 1)

        def flush(so):
            pltpu.async_copy(stg, rows_h.at[stgb], sem2).wait()
            for q in range(STG // L):
                stgb[pl.ds(q * L, L)] = jnp.full((L,), B, jnp.int32)
            return jnp.int32(0)

        for q in range(STG // L):
            stgb[pl.ds(q * L, L)] = jnp.full((L,), B, jnp.int32)

        def slab_one(c, buf, so):
            pltpu.make_async_copy(tabT.at[:, pl.ds(0, TW)], slab.at[buf],
                                  sems[buf]).wait()
            start = (slab0 + c) * TW

            def ev_body(v, so):
                so = lax.cond(so >= STG - L, flush, lambda x: x, so)
                r = er_v[pl.ds(v * L, L)]
                b = eb_v[pl.ds(v * L, L)]
                valid = lane_iota < (cnt - v * L)
                m = (r >= start) & (r < start + TW) & valid
                nh = lax.reduce_max(plsc.all_reduce_population_count(m), (0,))
                pos = plsc.cumsum(m.astype(jnp.int32)) - 1

                @pl.when(nh > 0)
                def _():
                    plsc.store_scatter(stgb, [pos + so], b, mask=m)
                    hcol = jnp.where(m, r - start, 0)

                    def hit_body(h, so2):
                        col = lax.reduce_max(
                            jnp.where(pos == h, hcol, 0), (0,))
                        colv = jnp.broadcast_to(col, (L,))
                        for j in range(D // L):
                            rows16 = lane_iota + j * L
                            stg[so2, pl.ds(j * L, L)] = plsc.load_gather(
                                slab.at[buf], [rows16, colv])
                        return so2 + 1
                    lax.fori_loop(0, nh, hit_body, so)
                return so + nh

            so = lax.fori_loop(0, nev, ev_body, so)

            @pl.when(c + 2 < my_n)
            def _():
                fire(c + 2, buf)
            return so

        def pair_body(p, so):
            for buf in range(2):
                c = p * 2 + buf
                so = lax.cond(
                    c < my_n,
                    functools.partial(slab_one, c, buf),
                    lambda x: x,
                    so,
                )
            return so

        so = lax.fori_loop(0, (my_n + 1) // 2, pair_body, jnp.int32(0))
        flush(so)

    return kern


def _make_combine(B, D):
    b_per_w = B // NW
    HB = b_per_w // 2

    @functools.partial(
        pl.kernel,
        out_type=jax.ShapeDtypeStruct((B,), jnp.float32),
        mesh=_MESH,
        compiler_params=_CPARAMS,
        scratch_types=[
            pltpu.VMEM((HB, TW), jnp.float32),
            pltpu.VMEM((HB, TW), jnp.float32),
            pltpu.VMEM((D,), jnp.float32),
            pltpu.VMEM((L,), jnp.float32),
            pltpu.VMEM((b_per_w,), jnp.float32),
        ],
    )
    def kern(urows_h, srows_h, w_h, bvec_h, out_h, ur, sr, wv, bvec, out_v):
        wid = lax.axis_index("s") * NC + lax.axis_index("c")
        base = wid * b_per_w
        pltpu.sync_copy(w_h, wv)
        pltpu.sync_copy(bvec_h, bvec)
        wregs = [wv[pl.ds(j * L, L)] for j in range(D // L)]
        lane_iota = lax.iota(jnp.int32, L)

        for half in range(2):
            hbase = base + half * HB
            pltpu.sync_copy(urows_h.at[pl.ds(hbase, HB)], ur)
            pltpu.sync_copy(srows_h.at[pl.ds(hbase, HB)], sr)

            def g_body(g, carry, half=half):
                acc = bvec[:]
                row0 = g * L
                vals = []
                for l in range(L):
                    p = jnp.zeros((L,), jnp.float32)
                    for j in range(D // L):
                        u = ur[row0 + l, pl.ds(j * L, L)]
                        s = sr[row0 + l, pl.ds(j * L, L)]
                        p = p + u * s * wregs[j]
                    vals.append(jnp.sum(p))
                res = bvec[:]
                for l in range(L):
                    res = jnp.where(lane_iota == l, vals[l], res)
                out_v[pl.ds(half * HB + g * L, L)] = res
                return carry

            lax.fori_loop(0, HB // L, g_body, 0)
        pltpu.sync_copy(out_v, out_h.at[pl.ds(base, b_per_w)])

    return kern


def kernel(user, song, user_embedding, song_embedding, fc_w, fc_b):
    B = user.shape[0]
    N, D = user_embedding.shape
    extract = _make_extract(B, N, D)
    combine = _make_combine(B, D)
    urows = extract(user.astype(jnp.int32), user_embedding.T)
    srows = extract(song.astype(jnp.int32), song_embedding.T)
    return combine(
        urows, srows,
        fc_w.reshape(D).astype(jnp.float32),
        jnp.broadcast_to(fc_b.reshape(1), (L,)).astype(jnp.float32),
    )


# speedup vs baseline: 1.3846x; 1.3846x over previous
import functools

import jax
import jax.numpy as jnp
from jax import lax
from jax.experimental import pallas as pl
from jax.experimental.pallas import tpu as pltpu
from jax.experimental.pallas import tpu_sc as plsc

NC = 2
NS = 16
L = 16
NW = NC * NS
TW = 128
STG = 256
PAD = 16

_CPARAMS = pltpu.CompilerParams(
    needs_layout_passes=False, use_tc_tiling_on_sc=True
)
_MESH = plsc.VectorSubcoreMesh(core_axis_name="c", subcore_axis_name="s")


def _make_extract(B, N, D):
    n_slab = (N + TW - 1) // TW
    base_per_w = n_slab // NW
    extra = n_slab - base_per_w * NW
    nvec = B // L

    @functools.partial(
        pl.kernel,
        out_type=jax.ShapeDtypeStruct((B + PAD, TW), jnp.float32),
        mesh=_MESH,
        compiler_params=_CPARAMS,
        scratch_types=[
            pltpu.VMEM((B + L,), jnp.int32),
            pltpu.VMEM((B + L,), jnp.int32),
            pltpu.VMEM((B + L,), jnp.int32),
            pltpu.VMEM((B + L,), jnp.int32),
            pltpu.VMEM((2, D, TW), jnp.float32),
            pltpu.VMEM((STG, TW), jnp.float32),
            pltpu.VMEM((STG,), jnp.int32),
            pltpu.SemaphoreType.DMA,
            pltpu.SemaphoreType.DMA,
            pltpu.SemaphoreType.DMA,
        ],
    )
    def kern(idx_h, tabT, rows_h, idx_v, er_v, eb_v, eb2, slab, stg,
             stgb, sem0, sem1, sem2):
        er2 = idx_v
        wid = lax.axis_index("s") * NC + lax.axis_index("c")
        my_n = base_per_w + jnp.where(wid < extra, 1, 0)
        slab0 = wid * base_per_w + jnp.minimum(wid, extra)
        lo = slab0 * TW
        sems = (sem0, sem1)

        pltpu.sync_copy(idx_h, idx_v.at[pl.ds(0, B)])
        lane_iota = lax.iota(jnp.int32, L)

        hi = lo + my_n * TW
        def collect(v, cnt):
            r = idx_v[pl.ds(v * L, L)]
            b = lane_iota + v * L
            m = (r >= lo) & (r < hi)
            plsc.store_compressed(er_v.at[pl.ds(cnt, L)], r, mask=m)
            plsc.store_compressed(eb_v.at[pl.ds(cnt, L)], b, mask=m)
            nh = lax.reduce_max(plsc.all_reduce_population_count(m), (0,))
            return cnt + nh
        cnt = lax.fori_loop(0, nvec, collect, jnp.int32(0))
        nev = (cnt + (L - 1)) // L

        NBK = 16
        BSL = 16
        offv_lo = jnp.zeros((L,), jnp.int32)
        offv_hi = jnp.zeros((L,), jnp.int32)
        off = jnp.int32(0)
        for kb in range(NBK):
            blo = lo + kb * (BSL * TW)
            bhi = blo + BSL * TW
            offv_lo = jnp.where(lane_iota == kb, off, offv_lo)

            def place(v, off, blo=blo, bhi=bhi):
                r = er_v[pl.ds(v * L, L)]
                b = eb_v[pl.ds(v * L, L)]
                valid = lane_iota < (cnt - v * L)
                m = (r >= blo) & (r < bhi) & valid
                plsc.store_compressed(er2.at[pl.ds(off, L)], r, mask=m)
                plsc.store_compressed(eb2.at[pl.ds(off, L)], b, mask=m)
                nh = lax.reduce_max(
                    plsc.all_reduce_population_count(m), (0,))
                return off + nh

            off = lax.fori_loop(0, nev, place, off)
            offv_hi = jnp.where(lane_iota == kb, off, offv_hi)

        def fire(c, buf):
            start = pl.multiple_of((slab0 + c) * TW, TW)
            pltpu.async_copy(tabT.at[:, pl.ds(start, TW)], slab.at[buf],
                             sems[buf])

        @pl.when(my_n > 0)
        def _():
            fire(0, 0)

        @pl.when(my_n > 1)
        def _():
            fire(1, 1)

        def flush(so):
            pltpu.async_copy(stg, rows_h.at[stgb], sem2).wait()
            for q in range(STG // L):
                stgb[pl.ds(q * L, L)] = jnp.full((L,), B, jnp.int32)
            return jnp.int32(0)

        for q in range(STG // L):
            stgb[pl.ds(q * L, L)] = jnp.full((L,), B, jnp.int32)

        def slab_one(c, buf, so):
            pltpu.make_async_copy(tabT.at[:, pl.ds(0, TW)], slab.at[buf],
                                  sems[buf]).wait()
            start = (slab0 + c) * TW
            kb = c // BSL
            e_lo = lax.reduce_max(
                jnp.where(lane_iota == kb, offv_lo, 0), (0,))
            e_hi = lax.reduce_max(
                jnp.where(lane_iota == kb, offv_hi, 0), (0,))
            nev_b = (e_hi - e_lo + (L - 1)) // L

            def ev_body(v, so):
                so = lax.cond(so >= STG - L, flush, lambda x: x, so)
                ebase = e_lo + v * L
                r = er2[pl.ds(ebase, L)]
                b = eb2[pl.ds(ebase, L)]
                valid = lane_iota < (e_hi - ebase)
                m = (r >= start) & (r < start + TW) & valid
                nh = lax.reduce_max(plsc.all_reduce_population_count(m), (0,))
                pos = plsc.cumsum(m.astype(jnp.int32)) - 1

                @pl.when(nh > 0)
                def _():
                    plsc.store_scatter(stgb, [pos + so], b, mask=m)
                    hcol = jnp.where(m, r - start, 0)

                    def hit_body(h, so2):
                        col = lax.reduce_max(
                            jnp.where(pos == h, hcol, 0), (0,))
                        colv = jnp.broadcast_to(col, (L,))
                        for j in range(D // L):
                            rows16 = lane_iota + j * L
                            stg[so2, pl.ds(j * L, L)] = plsc.load_gather(
                                slab.at[buf], [rows16, colv])
                        return so2 + 1
                    lax.fori_loop(0, nh, hit_body, so)
                return so + nh

            so = lax.fori_loop(0, nev_b, ev_body, so)

            @pl.when(c + 2 < my_n)
            def _():
                fire(c + 2, buf)
            return so

        def pair_body(p, so):
            for buf in range(2):
                c = p * 2 + buf
                so = lax.cond(
                    c < my_n,
                    functools.partial(slab_one, c, buf),
                    lambda x: x,
                    so,
                )
            return so

        so = lax.fori_loop(0, (my_n + 1) // 2, pair_body, jnp.int32(0))
        flush(so)

    return kern


def _make_combine(B, D):
    b_per_w = B // NW
    HB = b_per_w // 2

    @functools.partial(
        pl.kernel,
        out_type=jax.ShapeDtypeStruct((B,), jnp.float32),
        mesh=_MESH,
        compiler_params=_CPARAMS,
        scratch_types=[
            pltpu.VMEM((HB, TW), jnp.float32),
            pltpu.VMEM((HB, TW), jnp.float32),
            pltpu.VMEM((D,), jnp.float32),
            pltpu.VMEM((L,), jnp.float32),
            pltpu.VMEM((b_per_w,), jnp.float32),
        ],
    )
    def kern(urows_h, srows_h, w_h, bvec_h, out_h, ur, sr, wv, bvec, out_v):
        wid = lax.axis_index("s") * NC + lax.axis_index("c")
        base = wid * b_per_w
        pltpu.sync_copy(w_h, wv)
        pltpu.sync_copy(bvec_h, bvec)
        wregs = [wv[pl.ds(j * L, L)] for j in range(D // L)]
        lane_iota = lax.iota(jnp.int32, L)

        for half in range(2):
            hbase = base + half * HB
            pltpu.sync_copy(urows_h.at[pl.ds(hbase, HB)], ur)
            pltpu.sync_copy(srows_h.at[pl.ds(hbase, HB)], sr)

            def g_body(g, carry, half=half):
                acc = bvec[:]
                row0 = g * L
                vals = []
                for l in range(L):
                    p = jnp.zeros((L,), jnp.float32)
                    for j in range(D // L):
                        u = ur[row0 + l, pl.ds(j * L, L)]
                        s = sr[row0 + l, pl.ds(j * L, L)]
                        p = p + u * s * wregs[j]
                    vals.append(jnp.sum(p))
                res = bvec[:]
                for l in range(L):
                    res = jnp.where(lane_iota == l, vals[l], res)
                out_v[pl.ds(half * HB + g * L, L)] = res
                return carry

            lax.fori_loop(0, HB // L, g_body, 0)
        pltpu.sync_copy(out_v, out_h.at[pl.ds(base, b_per_w)])

    return kern


def kernel(user, song, user_embedding, song_embedding, fc_w, fc_b):
    B = user.shape[0]
    N, D = user_embedding.shape
    extract = _make_extract(B, N, D)
    combine = _make_combine(B, D)
    urows = extract(user.astype(jnp.int32), user_embedding.T)
    srows = extract(song.astype(jnp.int32), song_embedding.T)
    return combine(
        urows, srows,
        fc_w.reshape(D).astype(jnp.float32),
        jnp.broadcast_to(fc_b.reshape(1), (L,)).astype(jnp.float32),
    )


# speedup vs baseline: 3.1673x; 2.2874x over previous
import functools

import jax
import jax.numpy as jnp
from jax import lax
from jax.experimental import pallas as pl
from jax.experimental.pallas import tpu as pltpu
from jax.experimental.pallas import tpu_sc as plsc

NC = 2
NS = 16
L = 16
NW = NC * NS
NBUF = 4
TW = 128


def _make_kernel(B, D):
    b_per_w = B // NW
    mesh = plsc.VectorSubcoreMesh(core_axis_name="c", subcore_axis_name="s")

    @functools.partial(
        pl.kernel,
        out_type=jax.ShapeDtypeStruct((B,), jnp.float32),
        mesh=mesh,
        compiler_params=pltpu.CompilerParams(
            needs_layout_passes=False, use_tc_tiling_on_sc=True
        ),
        scratch_types=[
            pltpu.VMEM((b_per_w,), jnp.int32),
            pltpu.VMEM((b_per_w,), jnp.int32),
            pltpu.SMEM((b_per_w,), jnp.int32),
            pltpu.SMEM((b_per_w,), jnp.int32),
            pltpu.VMEM((NBUF, D, TW), jnp.float32),
            pltpu.VMEM((NBUF, D, TW), jnp.float32),
            pltpu.VMEM((D,), jnp.float32),
            pltpu.VMEM((L,), jnp.float32),
            pltpu.VMEM((b_per_w,), jnp.float32),
        ]
        + [pltpu.SemaphoreType.DMA] * NBUF,
    )
    def kern(user_h, song_h, uembT, sembT, w_h, bvec_h, out_h,
             uidx_v, sidx_v, uidx_s, sidx_s, ublk, sblk, wv, bvec, out_v,
             *sems):
        wid = lax.axis_index("s") * NC + lax.axis_index("c")
        base = wid * b_per_w

        pltpu.sync_copy(user_h.at[pl.ds(base, b_per_w)], uidx_v)
        pltpu.sync_copy(song_h.at[pl.ds(base, b_per_w)], sidx_v)
        pltpu.sync_copy(w_h, wv)
        pltpu.sync_copy(bvec_h, bvec)

        wregs = [wv[pl.ds(j * L, L)] for j in range(D // L)]
        lane_iota = lax.iota(jnp.int32, L)

        def ext_body(g, carry):
            uc = uidx_v[pl.ds(g * L, L)]
            sc = sidx_v[pl.ds(g * L, L)]
            for l in range(L):
                onehot = lane_iota == l
                uidx_s[g * L + l] = jnp.sum(jnp.where(onehot, uc, 0))
                sidx_s[g * L + l] = jnp.sum(jnp.where(onehot, sc, 0))
            return carry

        lax.fori_loop(0, b_per_w // L, ext_body, 0)

        def fetch(e, slot):
            ru = uidx_s[e]
            rs = sidx_s[e]
            cu = pl.multiple_of((ru // TW) * TW, TW)
            cs = pl.multiple_of((rs // TW) * TW, TW)
            pltpu.async_copy(uembT.at[:, pl.ds(cu, TW)], ublk.at[slot],
                             sems[slot])
            pltpu.async_copy(sembT.at[:, pl.ds(cs, TW)], sblk.at[slot],
                             sems[slot])

        for s in range(NBUF):
            fetch(s, s)

        def body(g, acc):
            for slot in range(NBUF):
                e = g * NBUF + slot
                pltpu.make_async_copy(uembT.at[:, pl.ds(0, TW)],
                                      ublk.at[slot], sems[slot]).wait()
                pltpu.make_async_copy(sembT.at[:, pl.ds(0, TW)],
                                      sblk.at[slot], sems[slot]).wait()
                lane_u = jnp.broadcast_to(uidx_s[e] % TW, (L,))
                lane_s = jnp.broadcast_to(sidx_s[e] % TW, (L,))
                p = jnp.zeros((L,), jnp.float32)
                for j in range(D // L):
                    rows = lane_iota + j * L
                    u = plsc.load_gather(ublk.at[slot], [rows, lane_u])
                    s = plsc.load_gather(sblk.at[slot], [rows, lane_s])
                    p = p + u * s * wregs[j]
                val = jnp.sum(p)

                @pl.when(e + NBUF < b_per_w)
                def _():
                    fetch(e + NBUF, slot)

                acc = jnp.where(lane_iota == e % L, val, acc)

                @pl.when(e % L == L - 1)
                def _():
                    out_v[pl.ds((e // L) * L, L)] = acc + bvec[:]
                acc = jnp.where(e % L == L - 1,
                                jnp.zeros((L,), jnp.float32), acc)
            return acc

        lax.fori_loop(0, b_per_w // NBUF, body,
                      jnp.zeros((L,), jnp.float32))
        pltpu.sync_copy(out_v, out_h.at[pl.ds(base, b_per_w)])

    return kern


def kernel(user, song, user_embedding, song_embedding, fc_w, fc_b):
    B = user.shape[0]
    D = user_embedding.shape[1]
    kern = _make_kernel(B, D)
    return kern(
        user.astype(jnp.int32),
        song.astype(jnp.int32),
        user_embedding.T,
        song_embedding.T,
        fc_w.reshape(D).astype(jnp.float32),
        jnp.broadcast_to(fc_b.reshape(1), (L,)).astype(jnp.float32),
    )


# speedup vs baseline: 3.1816x; 1.0045x over previous
import functools

import jax
import jax.numpy as jnp
from jax import lax
from jax.experimental import pallas as pl
from jax.experimental.pallas import tpu as pltpu
from jax.experimental.pallas import tpu_sc as plsc

NC = 2
NS = 16
L = 16
NW = NC * NS
NBUF = 4
TW = 128


def _make_kernel(B, D):
    b_per_w = B // NW
    mesh = plsc.VectorSubcoreMesh(core_axis_name="c", subcore_axis_name="s")

    @functools.partial(
        pl.kernel,
        out_type=jax.ShapeDtypeStruct((B,), jnp.float32),
        mesh=mesh,
        compiler_params=pltpu.CompilerParams(
            needs_layout_passes=False, use_tc_tiling_on_sc=True
        ),
        scratch_types=[
            pltpu.VMEM((b_per_w,), jnp.int32),
            pltpu.VMEM((b_per_w,), jnp.int32),
            pltpu.SMEM((b_per_w,), jnp.int32),
            pltpu.SMEM((b_per_w,), jnp.int32),
            pltpu.VMEM((NBUF, D, TW), jnp.float32),
            pltpu.VMEM((NBUF, D, TW), jnp.float32),
            pltpu.VMEM((D,), jnp.float32),
            pltpu.VMEM((L,), jnp.float32),
            pltpu.VMEM((b_per_w,), jnp.float32),
        ]
        + [pltpu.SemaphoreType.DMA] * NBUF,
    )
    def kern(user_h, song_h, uembT, sembT, w_h, bvec_h, out_h,
             uidx_v, sidx_v, uidx_s, sidx_s, ublk, sblk, wv, bvec, out_v,
             *sems):
        wid = lax.axis_index("s") * NC + lax.axis_index("c")
        base = wid * b_per_w

        pltpu.sync_copy(user_h.at[pl.ds(base, b_per_w)], uidx_v)
        pltpu.sync_copy(song_h.at[pl.ds(base, b_per_w)], sidx_v)
        pltpu.sync_copy(w_h, wv)
        pltpu.sync_copy(bvec_h, bvec)

        wregs = [wv[pl.ds(j * L, L)] for j in range(D // L)]
        lane_iota = lax.iota(jnp.int32, L)

        def ext_body(g, carry):
            uc = uidx_v[pl.ds(g * L, L)]
            sc = sidx_v[pl.ds(g * L, L)]
            for l in range(L):
                onehot = lane_iota == l
                uidx_s[g * L + l] = jnp.sum(jnp.where(onehot, uc, 0))
                sidx_s[g * L + l] = jnp.sum(jnp.where(onehot, sc, 0))
            return carry

        lax.fori_loop(0, b_per_w // L, ext_body, 0)

        def fetch(e, slot):
            ru = uidx_s[e]
            rs = sidx_s[e]
            cu = pl.multiple_of((ru // TW) * TW, TW)
            cs = pl.multiple_of((rs // TW) * TW, TW)
            half = 32
            for h in range(2):
                rows = pl.ds(h * half, half)
                pltpu.async_copy(uembT.at[rows, pl.ds(cu, TW)],
                                 ublk.at[slot, rows], sems[slot])
                pltpu.async_copy(sembT.at[rows, pl.ds(cs, TW)],
                                 sblk.at[slot, rows], sems[slot])

        for s in range(NBUF):
            fetch(s, s)

        def body(g, acc):
            for slot in range(NBUF):
                e = g * NBUF + slot
                pltpu.make_async_copy(uembT.at[:, pl.ds(0, TW)],
                                      ublk.at[slot], sems[slot]).wait()
                pltpu.make_async_copy(sembT.at[:, pl.ds(0, TW)],
                                      sblk.at[slot], sems[slot]).wait()
                lane_u = jnp.broadcast_to(uidx_s[e] % TW, (L,))
                lane_s = jnp.broadcast_to(sidx_s[e] % TW, (L,))
                p = jnp.zeros((L,), jnp.float32)
                for j in range(D // L):
                    rows = lane_iota + j * L
                    u = plsc.load_gather(ublk.at[slot], [rows, lane_u])
                    s = plsc.load_gather(sblk.at[slot], [rows, lane_s])
                    p = p + u * s * wregs[j]
                val = jnp.sum(p)

                @pl.when(e + NBUF < b_per_w)
                def _():
                    fetch(e + NBUF, slot)

                acc = jnp.where(lane_iota == e % L, val, acc)

                @pl.when(e % L == L - 1)
                def _():
                    out_v[pl.ds((e // L) * L, L)] = acc + bvec[:]
                acc = jnp.where(e % L == L - 1,
                                jnp.zeros((L,), jnp.float32), acc)
            return acc

        lax.fori_loop(0, b_per_w // NBUF, body,
                      jnp.zeros((L,), jnp.float32))
        pltpu.sync_copy(out_v, out_h.at[pl.ds(base, b_per_w)])

    return kern


def kernel(user, song, user_embedding, song_embedding, fc_w, fc_b):
    B = user.shape[0]
    D = user_embedding.shape[1]
    kern = _make_kernel(B, D)
    return kern(
        user.astype(jnp.int32),
        song.astype(jnp.int32),
        user_embedding.T,
        song_embedding.T,
        fc_w.reshape(D).astype(jnp.float32),
        jnp.broadcast_to(fc_b.reshape(1), (L,)).astype(jnp.float32),
    )
